# baseline (speedup 1.0000x reference)
import functools

import jax
import jax.numpy as jnp
from jax import lax
from jax.experimental import pallas as pl
from jax.experimental.pallas import tpu as pltpu
from jax.experimental.pallas import tpu_sc as plsc

B = 16384
NU = 1000000
NM = 100000
DU = 96
DM = 64
H = 32
DP = 128

NC = 2
NS = 16
NW = NC * NS
BPW = B // NW
CH = 128
NCH = BPW // CH

BLK = 2048
TRN = 8192
GRID_U = 31
GRID_M = 4
QU = GRID_U * TRN
QM = GRID_M * TRN


def _transform_body(x0_ref, x1_ref, x2_ref, x3_ref, w_ref, o_ref):
    for k, x_ref in enumerate((x0_ref, x1_ref, x2_ref, x3_ref)):
        y = jax.lax.dot_general(
            x_ref[...], w_ref[...], (((0,), (0,)), ((), ())),
            preferred_element_type=jnp.float32)
        o_ref[:, k * H:(k + 1) * H] = y.astype(jnp.bfloat16)


def _transform(table_t, w, grid):
    d, n = table_t.shape
    last = (n - 1) // TRN

    def idx(k, i):
        return (0, jnp.minimum(grid * k + i, last))

    specs = [
        pl.BlockSpec((d, TRN), functools.partial(idx, k)) for k in range(4)
    ]
    return pl.pallas_call(
        _transform_body,
        grid=(grid,),
        in_specs=specs + [pl.BlockSpec((d, H), lambda i: (0, 0))],
        out_specs=pl.BlockSpec((TRN, DP), lambda i: (i, 0)),
        out_shape=jax.ShapeDtypeStruct((grid * TRN, DP), jnp.bfloat16),
    )(table_t, table_t, table_t, table_t, w)


def _sc_gather(tu4, tm4, users2d, movies2d):
    mesh = plsc.VectorSubcoreMesh(core_axis_name="c", subcore_axis_name="s")

    @functools.partial(
        pl.kernel,
        mesh=mesh,
        compiler_params=pltpu.CompilerParams(use_tc_tiling_on_sc=False),
        out_type=[
            jax.ShapeDtypeStruct((B, DP), jnp.bfloat16),
            jax.ShapeDtypeStruct((B, DP), jnp.bfloat16),
        ],
        scratch_types=[
            pltpu.VMEM((NCH, CH), jnp.int32),
            pltpu.VMEM((NCH, CH), jnp.int32),
            pltpu.VMEM((2, CH, DP), jnp.bfloat16),
            pltpu.VMEM((2, CH, DP), jnp.bfloat16),
            pltpu.SemaphoreType.DMA,
            pltpu.SemaphoreType.DMA,
        ],
    )
    def k(ut_hbm, mt_hbm, ui_hbm, mi_hbm, u_out, m_out,
          ui_v, mi_v, ru_v, rm_v, gsem, osem):
        wid = lax.axis_index("s") * NC + lax.axis_index("c")
        base = wid * BPW
        pltpu.sync_copy(ui_hbm.at[pl.ds(wid * NCH, NCH)], ui_v)
        pltpu.sync_copy(mi_hbm.at[pl.ds(wid * NCH, NCH)], mi_v)
        gathers = [None, None]

        def issue(j, slot):
            gathers[slot] = (
                pltpu.async_copy(ut_hbm.at[ui_v.at[j]], ru_v.at[slot], gsem),
                pltpu.async_copy(mt_hbm.at[mi_v.at[j]], rm_v.at[slot], gsem),
            )

        issue(0, 0)
        outs = []
        for j in range(NCH):
            slot = j % 2
            for o in outs:
                o.wait()
            if j + 1 < NCH:
                issue(j + 1, 1 - slot)
            gu, gm = gathers[slot]
            gu.wait()
            gm.wait()
            outs = [
                pltpu.async_copy(
                    ru_v.at[slot], u_out.at[pl.ds(base + j * CH, CH)], osem),
                pltpu.async_copy(
                    rm_v.at[slot], m_out.at[pl.ds(base + j * CH, CH)], osem),
            ]
        for o in outs:
            o.wait()

    return k(tu4, tm4, users2d, movies2d)


def _select_group(x4, r):
    out = x4[:, 0:H]
    for k in (1, 2, 3):
        out = jnp.where(r == k, x4[:, k * H:(k + 1) * H], out)
    return out


def _mlp_body(u4_ref, m4_ref, ru_ref, rm_ref, b1_ref, w2_ref, b2_ref,
              w3_ref, b3_ref, o_ref):
    u = _select_group(u4_ref[...].astype(jnp.float32), ru_ref[...])
    m = _select_group(m4_ref[...].astype(jnp.float32), rm_ref[...])
    x = jnp.maximum(u + m + b1_ref[...], 0.0)
    x = jnp.dot(x, w2_ref[...], preferred_element_type=jnp.float32) + b2_ref[...]
    x = jnp.maximum(x, 0.0)
    o_ref[...] = (jnp.dot(x, w3_ref[...], preferred_element_type=jnp.float32)
                  + b3_ref[...])


def _tc_mlp(u4, m4, ru, rm, b1, W2, b2, W3, b3):
    full = lambda i: (0, 0)
    return pl.pallas_call(
        _mlp_body,
        grid=(B // BLK,),
        in_specs=[
            pl.BlockSpec((BLK, DP), lambda i: (i, 0)),
            pl.BlockSpec((BLK, DP), lambda i: (i, 0)),
            pl.BlockSpec((BLK, 1), lambda i: (i, 0)),
            pl.BlockSpec((BLK, 1), lambda i: (i, 0)),
            pl.BlockSpec((1, H), full),
            pl.BlockSpec((H, H), full),
            pl.BlockSpec((1, H), full),
            pl.BlockSpec((H, 1), full),
            pl.BlockSpec((1, 1), full),
        ],
        out_specs=pl.BlockSpec((BLK, 1), lambda i: (i, 0)),
        out_shape=jax.ShapeDtypeStruct((B, 1), jnp.float32),
    )(u4, m4, ru, rm, b1, W2, b2, W3, b3)


def kernel(users, movies, user_table, movie_table, W1, b1, W2, b2, W3, b3):
    users = users.astype(jnp.int32)
    movies = movies.astype(jnp.int32)
    tu4 = _transform(jnp.transpose(user_table), W1[:DU], GRID_U)
    tm4 = _transform(jnp.transpose(movie_table), W1[DU:], GRID_M)
    users2d = jnp.reshape(users % QU, (NW * NCH, CH))
    movies2d = jnp.reshape(movies % QM, (NW * NCH, CH))
    u4, m4 = _sc_gather(tu4, tm4, users2d, movies2d)
    ru = jnp.reshape(users // QU, (B, 1))
    rm = jnp.reshape(movies // QM, (B, 1))
    return _tc_mlp(u4, m4, ru, rm,
                   jnp.reshape(b1, (1, H)), W2, jnp.reshape(b2, (1, H)),
                   W3, jnp.reshape(b3, (1, 1)))


# speedup vs baseline: 1.7598x; 1.7598x over previous
import functools

import jax
import jax.numpy as jnp
from jax import lax
from jax.experimental import pallas as pl
from jax.experimental.pallas import tpu as pltpu
from jax.experimental.pallas import tpu_sc as plsc

B = 16384
NU = 1000000
NM = 100000
DU = 96
DM = 64
H = 32
DP = 128

NC = 2
NS = 16
NW = NC * NS
BPW = B // NW
CH = 128
NCH = BPW // CH

BLK = 2048
TRN = 8192
GRID_U = 31
GRID_M = 4
QU = GRID_U * TRN
QM = GRID_M * TRN


def _transform_body(x0_ref, x1_ref, x2_ref, x3_ref, w_ref, o_ref):
    for k, x_ref in enumerate((x0_ref, x1_ref, x2_ref, x3_ref)):
        y = jax.lax.dot_general(
            x_ref[...], w_ref[...], (((0,), (0,)), ((), ())),
            preferred_element_type=jnp.float32)
        o_ref[:, k * H:(k + 1) * H] = y


def _transform(table_t, w, grid):
    d, n = table_t.shape
    last = (n - 1) // TRN

    def idx(k, i):
        return (0, jnp.minimum(grid * k + i, last))

    specs = [
        pl.BlockSpec((d, TRN), functools.partial(idx, k)) for k in range(4)
    ]
    return pl.pallas_call(
        _transform_body,
        grid=(grid,),
        in_specs=specs + [pl.BlockSpec((d, H), lambda i: (0, 0))],
        out_specs=pl.BlockSpec((TRN, DP), lambda i: (i, 0)),
        out_shape=jax.ShapeDtypeStruct((grid * TRN, DP), jnp.float32),
    )(table_t, table_t, table_t, table_t, w)


def _sc_gather(tu4, tm4, users2d, movies2d):
    mesh = plsc.VectorSubcoreMesh(core_axis_name="c", subcore_axis_name="s")

    @functools.partial(
        pl.kernel,
        mesh=mesh,
        compiler_params=pltpu.CompilerParams(use_tc_tiling_on_sc=False),
        out_type=[
            jax.ShapeDtypeStruct((B, DP), jnp.float32),
            jax.ShapeDtypeStruct((B, DP), jnp.float32),
        ],
        scratch_types=[
            pltpu.VMEM((NCH, CH), jnp.int32),
            pltpu.VMEM((NCH, CH), jnp.int32),
            pltpu.VMEM((2, CH, DP), jnp.float32),
            pltpu.VMEM((2, CH, DP), jnp.float32),
            pltpu.SemaphoreType.DMA,
            pltpu.SemaphoreType.DMA,
        ],
    )
    def k(ut_hbm, mt_hbm, ui_hbm, mi_hbm, u_out, m_out,
          ui_v, mi_v, ru_v, rm_v, gsem, osem):
        wid = lax.axis_index("s") * NC + lax.axis_index("c")
        base = wid * BPW
        pltpu.sync_copy(ui_hbm.at[pl.ds(wid * NCH, NCH)], ui_v)
        pltpu.sync_copy(mi_hbm.at[pl.ds(wid * NCH, NCH)], mi_v)
        gathers = [None, None]

        def issue(j, slot):
            gathers[slot] = (
                pltpu.async_copy(ut_hbm.at[ui_v.at[j]], ru_v.at[slot], gsem),
                pltpu.async_copy(mt_hbm.at[mi_v.at[j]], rm_v.at[slot], gsem),
            )

        issue(0, 0)
        outs = []
        for j in range(NCH):
            slot = j % 2
            for o in outs:
                o.wait()
            if j + 1 < NCH:
                issue(j + 1, 1 - slot)
            gu, gm = gathers[slot]
            gu.wait()
            gm.wait()
            outs = [
                pltpu.async_copy(
                    ru_v.at[slot], u_out.at[pl.ds(base + j * CH, CH)], osem),
                pltpu.async_copy(
                    rm_v.at[slot], m_out.at[pl.ds(base + j * CH, CH)], osem),
            ]
        for o in outs:
            o.wait()

    return k(tu4, tm4, users2d, movies2d)


def _select_group(x4, r):
    out = x4[:, 0:H]
    for k in (1, 2, 3):
        out = jnp.where(r == k, x4[:, k * H:(k + 1) * H], out)
    return out


def _mlp_body(u4_ref, m4_ref, ru_ref, rm_ref, b1_ref, w2_ref, b2_ref,
              w3_ref, b3_ref, o_ref):
    u = _select_group(u4_ref[...], ru_ref[...])
    m = _select_group(m4_ref[...], rm_ref[...])
    x = jnp.maximum(u + m + b1_ref[...], 0.0)
    x = jnp.dot(x, w2_ref[...], preferred_element_type=jnp.float32) + b2_ref[...]
    x = jnp.maximum(x, 0.0)
    o_ref[...] = (jnp.dot(x, w3_ref[...], preferred_element_type=jnp.float32)
                  + b3_ref[...])


def _tc_mlp(u4, m4, ru, rm, b1, W2, b2, W3, b3):
    full = lambda i: (0, 0)
    return pl.pallas_call(
        _mlp_body,
        grid=(B // BLK,),
        in_specs=[
            pl.BlockSpec((BLK, DP), lambda i: (i, 0)),
            pl.BlockSpec((BLK, DP), lambda i: (i, 0)),
            pl.BlockSpec((BLK, 1), lambda i: (i, 0)),
            pl.BlockSpec((BLK, 1), lambda i: (i, 0)),
            pl.BlockSpec((1, H), full),
            pl.BlockSpec((H, H), full),
            pl.BlockSpec((1, H), full),
            pl.BlockSpec((H, 1), full),
            pl.BlockSpec((1, 1), full),
        ],
        out_specs=pl.BlockSpec((BLK, 1), lambda i: (i, 0)),
        out_shape=jax.ShapeDtypeStruct((B, 1), jnp.float32),
    )(u4, m4, ru, rm, b1, W2, b2, W3, b3)


def kernel(users, movies, user_table, movie_table, W1, b1, W2, b2, W3, b3):
    users = users.astype(jnp.int32)
    movies = movies.astype(jnp.int32)
    tu4 = _transform(jnp.transpose(user_table), W1[:DU], GRID_U)
    tm4 = _transform(jnp.transpose(movie_table), W1[DU:], GRID_M)
    users2d = jnp.reshape(users % QU, (NW * NCH, CH))
    movies2d = jnp.reshape(movies % QM, (NW * NCH, CH))
    u4, m4 = _sc_gather(tu4, tm4, users2d, movies2d)
    ru = jnp.reshape(users // QU, (B, 1))
    rm = jnp.reshape(movies // QM, (B, 1))
    return _tc_mlp(u4, m4, ru, rm,
                   jnp.reshape(b1, (1, H)), W2, jnp.reshape(b2, (1, H)),
                   W3, jnp.reshape(b3, (1, 1)))
